# baseline (speedup 1.0000x reference)
import jax
import jax.numpy as jnp
from jax import lax
from jax.experimental import pallas as pl
from jax.experimental.pallas import tpu as pltpu
from jax.experimental.pallas import tpu_sc as plsc

N_NODES = 10000
N_EDGES = 160000
D = 256

NC = 2
NS = 16
L = 16

DH = D // 2
Q = N_NODES // 4
ACC_ROWS = 2 * Q + 120
DUMP = 2 * Q
E_T = N_EDGES // NS
NBUF = 8
G = 16
CAP = 10240
NVEC = E_T // L
NFILL = CAP // L
ZCOPY = ACC_ROWS // NS
ZROWS = 32
OUT_CHUNKS = 2 * Q // 8


def _body(src_hbm, dst_hbm, x_hbm, out_hbm,
          src_v, dst_v, srcc, dstc, gbuf, zbuf, cnt_v, acc,
          sg0, sg1, sg2, sg3, ss0, ss1, ss2, ss3, sem3):
    sems = (sg0, sg1, sg2, sg3, ss0, ss1, ss2, ss3)
    cid = lax.axis_index("c")
    sid = lax.axis_index("s")
    lo = cid * 2 * Q

    pltpu.sync_copy(src_hbm.at[pl.ds(sid * E_T, E_T)], src_v)
    pltpu.sync_copy(dst_hbm.at[pl.ds(sid * E_T, E_T)], dst_v)

    zf = jnp.zeros((L,), jnp.float32)

    def zero_row(r, carry):
        for j in range(DH // L):
            zbuf[r, pl.ds(j * L, L)] = zf
        return carry

    lax.fori_loop(0, ZROWS, zero_row, 0)
    zbase = sid * ZCOPY

    def zero_acc():
        for q in range(0, ZCOPY, ZROWS):
            n = min(ZROWS, ZCOPY - q)
            pltpu.sync_copy(zbuf.at[pl.ds(0, n)],
                            acc.at[pl.ds(zbase + q, n)])

    zero_acc()

    zi = jnp.zeros((L,), jnp.int32)
    iota = lax.broadcasted_iota(jnp.int32, (L,), 0)
    dump_lo = jnp.full((L,), DUMP, jnp.int32) + iota

    def fill_src(k, carry):
        srcc[pl.ds(k * L, L)] = zi
        return carry

    def fill_dst(k, carry):
        dstc[pl.ds(k * L, L)] = dump_lo + (k % 2) * L
        return carry

    lax.fori_loop(0, NFILL, fill_src, 0)
    lax.fori_loop(0, 2 * NFILL, fill_dst, 0)

    lo16 = jnp.full((L,), lo, jnp.int32)
    q16 = jnp.full((L,), Q, jnp.int32)
    one16 = jnp.full((L,), 1, jnp.int32)
    top16 = jnp.full((L,), CAP - 1, jnp.int32)

    def compact(i, ptrs):
        p0, p1 = ptrs
        s16 = src_v[pl.ds(i * L, L)]
        d16 = dst_v[pl.ds(i * L, L)]
        dl = d16 - lo16
        m0 = (dl >= 0) & (dl < q16)
        dl1 = dl - q16
        m1 = (dl1 >= 0) & (dl1 < q16)
        mi0 = jnp.where(m0, one16, zi)
        mi1 = jnp.where(m1, one16, zi)
        pos0 = p0 + plsc.cumsum(mi0) - mi0
        pos1 = top16 - (p1 + plsc.cumsum(mi1) - mi1)
        plsc.store_scatter(srcc, [pos0], s16, mask=m0)
        plsc.store_scatter(srcc, [pos1], s16, mask=m1)
        e0 = dl + dl
        e1 = dl1 + dl1
        plsc.store_scatter(dstc, [pos0 + pos0], e0, mask=m0)
        plsc.store_scatter(dstc, [pos0 + pos0 + one16], e0 + one16, mask=m0)
        plsc.store_scatter(dstc, [pos1 + pos1], e1, mask=m1)
        plsc.store_scatter(dstc, [pos1 + pos1 + one16], e1 + one16, mask=m1)
        return (p0 + plsc.all_reduce_population_count(m0),
                p1 + plsc.all_reduce_population_count(m1))

    ptr0, ptr1 = lax.fori_loop(
        0, NVEC, compact,
        (jnp.zeros((L,), jnp.int32), jnp.zeros((L,), jnp.int32)))
    cnt_v[pl.ds(0, L)] = ptr0
    cnt_v[pl.ds(L, L)] = ptr1
    cnt0 = cnt_v[pl.ds(0, L)][0]
    cnt1 = cnt_v[pl.ds(L, L)][0]

    plsc.subcore_barrier()

    for p in range(2):
        cnt = cnt0 if p == 0 else cnt1
        nchn = (cnt + (NBUF * G - 1)) // (NBUF * G)

        if p == 0:
            def off(c):
                return c * G
        else:
            def off(c):
                return CAP - G - c * G

        def start(c, b):
            pltpu.async_copy(
                x_hbm.at[srcc.at[pl.ds(off(c), G)]], gbuf.at[b], sems[b])

        def drain(b):
            pltpu.make_async_copy(
                x_hbm.at[pl.ds(0, G)], gbuf.at[b], sems[b]).wait()

        def scatter(c, b):
            idx = dstc.at[pl.ds(2 * off(c), 2 * G)]
            pltpu.sync_copy(gbuf.at[b].reshape(2 * G, DH),
                            acc.at[idx], add=True)

        @pl.when(nchn > 0)
        def _():
            for b in range(NBUF):
                start(b, b)

        def chunk_group(cc, carry):
            for b in range(NBUF):
                drain(b)
                scatter(NBUF * cc + b, b)

                @pl.when(cc + 1 < nchn)
                def _():
                    start(NBUF * (cc + 1) + b, b)

            return carry

        lax.fori_loop(0, nchn, chunk_group, 0)

        plsc.subcore_barrier()

        qlo2 = 2 * (lo + p * Q)

        def out_chunk(k, carry):
            j = sid + k * NS

            @pl.when(j < OUT_CHUNKS)
            def _():
                pltpu.async_copy(
                    acc.at[pl.ds(j * 8, 8)],
                    out_hbm.at[pl.ds(qlo2 + j * 8, 8)],
                    sem3)

            return carry

        def out_wait(k, carry):
            j = sid + k * NS

            @pl.when(j < OUT_CHUNKS)
            def _():
                pltpu.make_async_copy(
                    acc.at[pl.ds(0, 8)],
                    out_hbm.at[pl.ds(qlo2, 8)],
                    sem3).wait()

            return carry

        nk = (OUT_CHUNKS + NS - 1) // NS
        lax.fori_loop(0, nk, out_chunk, 0)
        lax.fori_loop(0, nk, out_wait, 0)

        if p == 0:
            plsc.subcore_barrier()
            zero_acc()
            plsc.subcore_barrier()


_seg_sum = pl.kernel(
    _body,
    out_type=jax.ShapeDtypeStruct((2 * N_NODES, DH), jnp.float32),
    mesh=plsc.VectorSubcoreMesh(
        core_axis_name="c", subcore_axis_name="s",
        num_cores=NC, num_subcores=NS),
    compiler_params=pltpu.CompilerParams(needs_layout_passes=False),
    scratch_types=[
        pltpu.VMEM((E_T,), jnp.int32),
        pltpu.VMEM((E_T,), jnp.int32),
        pltpu.VMEM((CAP,), jnp.int32),
        pltpu.VMEM((2 * CAP,), jnp.int32),
        pltpu.VMEM((NBUF, G, 2, DH), jnp.float32),
        pltpu.VMEM((ZROWS, DH), jnp.float32),
        pltpu.VMEM((2 * L,), jnp.int32),
        pltpu.VMEM_SHARED((ACC_ROWS, DH), jnp.float32),
        pltpu.SemaphoreType.DMA,
        pltpu.SemaphoreType.DMA,
        pltpu.SemaphoreType.DMA,
        pltpu.SemaphoreType.DMA,
        pltpu.SemaphoreType.DMA,
        pltpu.SemaphoreType.DMA,
        pltpu.SemaphoreType.DMA,
        pltpu.SemaphoreType.DMA,
        pltpu.SemaphoreType.DMA,
    ],
)


@jax.jit
def kernel(edge_index, mask, x):
    del mask
    src = edge_index[0]
    dst = edge_index[1]
    out2 = _seg_sum(src, dst, x.reshape(N_NODES, 2, DH))
    return out2.reshape(N_NODES, D)


# speedup vs baseline: 1.2963x; 1.2963x over previous
import jax
import jax.numpy as jnp
from jax import lax
from jax.experimental import pallas as pl
from jax.experimental.pallas import tpu as pltpu
from jax.experimental.pallas import tpu_sc as plsc

N_NODES = 10000
N_EDGES = 160000
D = 256

NC = 2
NS = 16
L = 16

DH = D // 2
Q = N_NODES // 4
ACC_ROWS = 2 * Q + 120
DUMP = 2 * Q
E_T = N_EDGES // NS
NBUF = 4
G = 16
CAP = 10240
NVEC = E_T // L
NFILL = CAP // L
ZCOPY = ACC_ROWS // NS
ZROWS = 32
OUT_CHUNKS = 2 * Q // 8


def _body(src_hbm, dst_hbm, x_hbm, out_hbm,
          src_v, dst_v, srcc, dstc, gbuf, zbuf, cnt_v, acc,
          sg0, sg1, sg2, sg3, ss0, ss1, ss2, ss3, sem3):
    sems = (sg0, sg1, sg2, sg3, ss0, ss1, ss2, ss3)
    cid = lax.axis_index("c")
    sid = lax.axis_index("s")
    lo = cid * 2 * Q

    pltpu.sync_copy(src_hbm.at[pl.ds(sid * E_T, E_T)], src_v)
    pltpu.sync_copy(dst_hbm.at[pl.ds(sid * E_T, E_T)], dst_v)

    zf = jnp.zeros((L,), jnp.float32)

    def zero_row(r, carry):
        for j in range(DH // L):
            zbuf[r, pl.ds(j * L, L)] = zf
        return carry

    lax.fori_loop(0, ZROWS, zero_row, 0)
    zbase = sid * ZCOPY

    def zero_acc():
        for q in range(0, ZCOPY, ZROWS):
            n = min(ZROWS, ZCOPY - q)
            pltpu.sync_copy(zbuf.at[pl.ds(0, n)],
                            acc.at[pl.ds(zbase + q, n)])

    zero_acc()

    zi = jnp.zeros((L,), jnp.int32)
    iota = lax.broadcasted_iota(jnp.int32, (L,), 0)
    dump_lo = jnp.full((L,), DUMP, jnp.int32) + iota

    lo16 = jnp.full((L,), lo, jnp.int32)
    q16 = jnp.full((L,), Q, jnp.int32)
    one16 = jnp.full((L,), 1, jnp.int32)
    top16 = jnp.full((L,), CAP - 1, jnp.int32)

    def compact(i, ptrs):
        p0, p1 = ptrs
        s16 = src_v[pl.ds(i * L, L)]
        d16 = dst_v[pl.ds(i * L, L)]
        dl = d16 - lo16
        m0 = (dl >= 0) & (dl < q16)
        dl1 = dl - q16
        m1 = (dl1 >= 0) & (dl1 < q16)
        mi0 = jnp.where(m0, one16, zi)
        mi1 = jnp.where(m1, one16, zi)
        pos0 = p0 + plsc.cumsum(mi0) - mi0
        pos1 = top16 - (p1 + plsc.cumsum(mi1) - mi1)
        plsc.store_scatter(srcc, [pos0], s16, mask=m0)
        plsc.store_scatter(srcc, [pos1], s16, mask=m1)
        e0 = dl + dl
        e1 = dl1 + dl1
        plsc.store_scatter(dstc, [pos0 + pos0], e0, mask=m0)
        plsc.store_scatter(dstc, [pos0 + pos0 + one16], e0 + one16, mask=m0)
        plsc.store_scatter(dstc, [pos1 + pos1], e1, mask=m1)
        plsc.store_scatter(dstc, [pos1 + pos1 + one16], e1 + one16, mask=m1)
        return (p0 + plsc.all_reduce_population_count(m0),
                p1 + plsc.all_reduce_population_count(m1))

    ptr0, ptr1 = lax.fori_loop(
        0, NVEC, compact,
        (jnp.zeros((L,), jnp.int32), jnp.zeros((L,), jnp.int32)))
    cnt_v[pl.ds(0, L)] = ptr0
    cnt_v[pl.ds(L, L)] = ptr1
    cnt0 = cnt_v[pl.ds(0, L)][0]
    cnt1 = cnt_v[pl.ds(L, L)][0]

    c0_16 = jnp.full((L,), cnt0, jnp.int32) + iota
    c1_16 = jnp.full((L,), CAP - cnt1 - NBUF * G, jnp.int32) + iota
    for k in range(NBUF * G // L):
        plsc.store_scatter(srcc, [c0_16 + k * L], zi)
        plsc.store_scatter(srcc, [c1_16 + k * L], zi)
    d0_16 = c0_16 + c0_16 - iota
    d1_16 = jnp.full((L,), 2 * (CAP - cnt1) - 2 * NBUF * G, jnp.int32) + iota
    for k in range(2 * NBUF * G // L):
        plsc.store_scatter(dstc, [d0_16 + k * L], dump_lo + (k % 2) * L)
        plsc.store_scatter(dstc, [d1_16 + k * L], dump_lo + (k % 2) * L)

    plsc.subcore_barrier()

    for p in range(2):
        cnt = cnt0 if p == 0 else cnt1
        nchn = (cnt + (NBUF * G - 1)) // (NBUF * G)

        if p == 0:
            def off(c):
                return c * G
        else:
            def off(c):
                return CAP - G - c * G

        def start(c, b):
            pltpu.async_copy(
                x_hbm.at[srcc.at[pl.ds(off(c), G)]], gbuf.at[b], sems[b])

        def drain(b):
            pltpu.make_async_copy(
                x_hbm.at[pl.ds(0, G)], gbuf.at[b], sems[b]).wait()

        def scatter(c, b):
            idx = dstc.at[pl.ds(2 * off(c), 2 * G)]
            pltpu.sync_copy(gbuf.at[b].reshape(2 * G, DH),
                            acc.at[idx], add=True)

        @pl.when(nchn > 0)
        def _():
            for b in range(NBUF):
                start(b, b)

        def chunk_group(cc, carry):
            for b in range(NBUF):
                drain(b)
                scatter(NBUF * cc + b, b)

                @pl.when(cc + 1 < nchn)
                def _():
                    start(NBUF * (cc + 1) + b, b)

            return carry

        lax.fori_loop(0, nchn, chunk_group, 0)

        plsc.subcore_barrier()

        qlo2 = 2 * (lo + p * Q)

        def out_chunk(k, carry):
            j = sid + k * NS

            @pl.when(j < OUT_CHUNKS)
            def _():
                pltpu.async_copy(
                    acc.at[pl.ds(j * 8, 8)],
                    out_hbm.at[pl.ds(qlo2 + j * 8, 8)],
                    sem3)

            return carry

        def out_wait(k, carry):
            j = sid + k * NS

            @pl.when(j < OUT_CHUNKS)
            def _():
                pltpu.make_async_copy(
                    acc.at[pl.ds(0, 8)],
                    out_hbm.at[pl.ds(qlo2, 8)],
                    sem3).wait()

            return carry

        nk = (OUT_CHUNKS + NS - 1) // NS
        lax.fori_loop(0, nk, out_chunk, 0)
        lax.fori_loop(0, nk, out_wait, 0)

        if p == 0:
            plsc.subcore_barrier()
            zero_acc()
            plsc.subcore_barrier()


_seg_sum = pl.kernel(
    _body,
    out_type=jax.ShapeDtypeStruct((2 * N_NODES, DH), jnp.float32),
    mesh=plsc.VectorSubcoreMesh(
        core_axis_name="c", subcore_axis_name="s",
        num_cores=NC, num_subcores=NS),
    compiler_params=pltpu.CompilerParams(needs_layout_passes=False),
    scratch_types=[
        pltpu.VMEM((E_T,), jnp.int32),
        pltpu.VMEM((E_T,), jnp.int32),
        pltpu.VMEM((CAP,), jnp.int32),
        pltpu.VMEM((2 * CAP,), jnp.int32),
        pltpu.VMEM((NBUF, G, 2, DH), jnp.float32),
        pltpu.VMEM((ZROWS, DH), jnp.float32),
        pltpu.VMEM((2 * L,), jnp.int32),
        pltpu.VMEM_SHARED((ACC_ROWS, DH), jnp.float32),
        pltpu.SemaphoreType.DMA,
        pltpu.SemaphoreType.DMA,
        pltpu.SemaphoreType.DMA,
        pltpu.SemaphoreType.DMA,
        pltpu.SemaphoreType.DMA,
        pltpu.SemaphoreType.DMA,
        pltpu.SemaphoreType.DMA,
        pltpu.SemaphoreType.DMA,
        pltpu.SemaphoreType.DMA,
    ],
)


@jax.jit
def kernel(edge_index, mask, x):
    del mask
    src = edge_index[0]
    dst = edge_index[1]
    out2 = _seg_sum(src, dst, x.reshape(N_NODES, 2, DH))
    return out2.reshape(N_NODES, D)


# speedup vs baseline: 1.3358x; 1.0305x over previous
import jax
import jax.numpy as jnp
from jax import lax
from jax.experimental import pallas as pl
from jax.experimental.pallas import tpu as pltpu
from jax.experimental.pallas import tpu_sc as plsc

N_NODES = 10000
N_EDGES = 160000
D = 256

NC = 2
NS = 16
L = 16

DH = D // 2
Q = N_NODES // 4
ACC_ROWS = 2 * Q + 120
DUMP = 2 * Q
E_T = N_EDGES // NS
NBUF = 4
G = 16
CAP = 10240
NVEC = E_T // L
NFILL = CAP // L
ZCOPY = ACC_ROWS // NS
ZROWS = 32
OUT_CHUNKS = 2 * Q // 8


def _body(src_hbm, dst_hbm, x_hbm, out_hbm,
          src_v, dst_v, srcc, dstc, gbuf, zbuf, cnt_v, acc,
          sg0, sg1, sg2, sg3, ss0, ss1, ss2, ss3, sem3):
    sems = (sg0, sg1, sg2, sg3, ss0, ss1, ss2, ss3)
    cid = lax.axis_index("c")
    sid = lax.axis_index("s")
    lo = cid * 2 * Q

    pltpu.sync_copy(src_hbm.at[pl.ds(sid * E_T, E_T)], src_v)
    pltpu.sync_copy(dst_hbm.at[pl.ds(sid * E_T, E_T)], dst_v)

    zf = jnp.zeros((L,), jnp.float32)

    def zero_row(r, carry):
        for j in range(DH // L):
            zbuf[r, pl.ds(j * L, L)] = zf
        return carry

    lax.fori_loop(0, ZROWS, zero_row, 0)
    zbase = sid * ZCOPY

    def zero_acc():
        for q in range(0, ZCOPY, ZROWS):
            n = min(ZROWS, ZCOPY - q)
            pltpu.sync_copy(zbuf.at[pl.ds(0, n)],
                            acc.at[pl.ds(zbase + q, n)])

    zero_acc()

    zi = jnp.zeros((L,), jnp.int32)
    iota = lax.broadcasted_iota(jnp.int32, (L,), 0)
    dump_lo = jnp.full((L,), DUMP, jnp.int32) + iota

    lo16 = jnp.full((L,), lo, jnp.int32)
    q16 = jnp.full((L,), Q, jnp.int32)
    one16 = jnp.full((L,), 1, jnp.int32)
    top16 = jnp.full((L,), CAP - 1, jnp.int32)

    @plsc.parallel_loop(
        0, NVEC, unroll=4,
        carry=(jnp.zeros((L,), jnp.int32), jnp.zeros((L,), jnp.int32)))
    def compact(i, ptrs):
        p0, p1 = ptrs
        s16 = src_v[pl.ds(i * L, L)]
        d16 = dst_v[pl.ds(i * L, L)]
        dl = d16 - lo16
        m0 = (dl >= 0) & (dl < q16)
        dl1 = dl - q16
        m1 = (dl1 >= 0) & (dl1 < q16)
        mi0 = jnp.where(m0, one16, zi)
        mi1 = jnp.where(m1, one16, zi)
        pos0 = p0 + plsc.cumsum(mi0) - mi0
        pos1 = top16 - (p1 + plsc.cumsum(mi1) - mi1)
        plsc.store_scatter(srcc, [pos0], s16, mask=m0)
        plsc.store_scatter(srcc, [pos1], s16, mask=m1)
        e0 = dl + dl
        e1 = dl1 + dl1
        plsc.store_scatter(dstc, [pos0 + pos0], e0, mask=m0)
        plsc.store_scatter(dstc, [pos0 + pos0 + one16], e0 + one16, mask=m0)
        plsc.store_scatter(dstc, [pos1 + pos1], e1, mask=m1)
        plsc.store_scatter(dstc, [pos1 + pos1 + one16], e1 + one16, mask=m1)
        return (p0 + plsc.all_reduce_population_count(m0),
                p1 + plsc.all_reduce_population_count(m1))

    ptr0, ptr1 = compact
    cnt_v[pl.ds(0, L)] = ptr0
    cnt_v[pl.ds(L, L)] = ptr1
    cnt0 = cnt_v[pl.ds(0, L)][0]
    cnt1 = cnt_v[pl.ds(L, L)][0]

    c0_16 = jnp.full((L,), cnt0, jnp.int32) + iota
    c1_16 = jnp.full((L,), CAP - cnt1 - NBUF * G, jnp.int32) + iota
    for k in range(NBUF * G // L):
        plsc.store_scatter(srcc, [c0_16 + k * L], zi)
        plsc.store_scatter(srcc, [c1_16 + k * L], zi)
    d0_16 = c0_16 + c0_16 - iota
    d1_16 = jnp.full((L,), 2 * (CAP - cnt1) - 2 * NBUF * G, jnp.int32) + iota
    for k in range(2 * NBUF * G // L):
        plsc.store_scatter(dstc, [d0_16 + k * L], dump_lo + (k % 2) * L)
        plsc.store_scatter(dstc, [d1_16 + k * L], dump_lo + (k % 2) * L)

    plsc.subcore_barrier()

    for p in range(2):
        cnt = cnt0 if p == 0 else cnt1
        nchn = (cnt + (NBUF * G - 1)) // (NBUF * G)

        if p == 0:
            def off(c):
                return c * G
        else:
            def off(c):
                return CAP - G - c * G

        def start(c, b):
            pltpu.async_copy(
                x_hbm.at[srcc.at[pl.ds(off(c), G)]], gbuf.at[b], sems[b])

        def drain(b):
            pltpu.make_async_copy(
                x_hbm.at[pl.ds(0, G)], gbuf.at[b], sems[b]).wait()

        def scatter(c, b):
            idx = dstc.at[pl.ds(2 * off(c), 2 * G)]
            pltpu.sync_copy(gbuf.at[b].reshape(2 * G, DH),
                            acc.at[idx], add=True)

        @pl.when(nchn > 0)
        def _():
            for b in range(NBUF):
                start(b, b)

        def chunk_group(cc, carry):
            for b in range(NBUF):
                drain(b)
                scatter(NBUF * cc + b, b)

                @pl.when(cc + 1 < nchn)
                def _():
                    start(NBUF * (cc + 1) + b, b)

            return carry

        lax.fori_loop(0, nchn, chunk_group, 0)

        plsc.subcore_barrier()

        qlo2 = 2 * (lo + p * Q)

        def out_chunk(k, carry):
            j = sid + k * NS

            @pl.when(j < OUT_CHUNKS)
            def _():
                pltpu.async_copy(
                    acc.at[pl.ds(j * 8, 8)],
                    out_hbm.at[pl.ds(qlo2 + j * 8, 8)],
                    sem3)

            return carry

        def out_wait(k, carry):
            j = sid + k * NS

            @pl.when(j < OUT_CHUNKS)
            def _():
                pltpu.make_async_copy(
                    acc.at[pl.ds(0, 8)],
                    out_hbm.at[pl.ds(qlo2, 8)],
                    sem3).wait()

            return carry

        nk = (OUT_CHUNKS + NS - 1) // NS
        lax.fori_loop(0, nk, out_chunk, 0)
        lax.fori_loop(0, nk, out_wait, 0)

        if p == 0:
            plsc.subcore_barrier()
            zero_acc()
            plsc.subcore_barrier()


_seg_sum = pl.kernel(
    _body,
    out_type=jax.ShapeDtypeStruct((2 * N_NODES, DH), jnp.float32),
    mesh=plsc.VectorSubcoreMesh(
        core_axis_name="c", subcore_axis_name="s",
        num_cores=NC, num_subcores=NS),
    compiler_params=pltpu.CompilerParams(needs_layout_passes=False),
    scratch_types=[
        pltpu.VMEM((E_T,), jnp.int32),
        pltpu.VMEM((E_T,), jnp.int32),
        pltpu.VMEM((CAP,), jnp.int32),
        pltpu.VMEM((2 * CAP,), jnp.int32),
        pltpu.VMEM((NBUF, G, 2, DH), jnp.float32),
        pltpu.VMEM((ZROWS, DH), jnp.float32),
        pltpu.VMEM((2 * L,), jnp.int32),
        pltpu.VMEM_SHARED((ACC_ROWS, DH), jnp.float32),
        pltpu.SemaphoreType.DMA,
        pltpu.SemaphoreType.DMA,
        pltpu.SemaphoreType.DMA,
        pltpu.SemaphoreType.DMA,
        pltpu.SemaphoreType.DMA,
        pltpu.SemaphoreType.DMA,
        pltpu.SemaphoreType.DMA,
        pltpu.SemaphoreType.DMA,
        pltpu.SemaphoreType.DMA,
    ],
)


@jax.jit
def kernel(edge_index, mask, x):
    del mask
    src = edge_index[0]
    dst = edge_index[1]
    out2 = _seg_sum(src, dst, x.reshape(N_NODES, 2, DH))
    return out2.reshape(N_NODES, D)


# speedup vs baseline: 1.3372x; 1.0011x over previous
import jax
import jax.numpy as jnp
from jax import lax
from jax.experimental import pallas as pl
from jax.experimental.pallas import tpu as pltpu
from jax.experimental.pallas import tpu_sc as plsc

N_NODES = 10000
N_EDGES = 160000
D = 256

NC = 2
NS = 16
L = 16

DH = D // 2
Q = N_NODES // 4
ACC_ROWS = 2 * Q + 120
DUMP = 2 * Q
E_T = N_EDGES // NS
NBUF = 4
G = 16
CAP = 10240
NVEC = E_T // L
NFILL = CAP // L
ZCOPY = ACC_ROWS // NS
ZROWS = 32
OUT_CHUNKS = 2 * Q // 8


def _body(src_hbm, dst_hbm, x_hbm, out_hbm,
          src_v, dst_v, srcc, dstc, gbuf, zbuf, cnt_v, acc,
          sg0, sg1, sg2, sg3, ss0, ss1, ss2, ss3, sem3):
    sems = (sg0, sg1, sg2, sg3, ss0, ss1, ss2, ss3)
    cid = lax.axis_index("c")
    sid = lax.axis_index("s")
    lo = cid * 2 * Q

    pltpu.sync_copy(src_hbm.at[pl.ds(sid * E_T, E_T)], src_v)
    pltpu.sync_copy(dst_hbm.at[pl.ds(sid * E_T, E_T)], dst_v)

    zf = jnp.zeros((L,), jnp.float32)

    def zero_row(r, carry):
        for j in range(DH // L):
            zbuf[r, pl.ds(j * L, L)] = zf
        return carry

    lax.fori_loop(0, ZROWS, zero_row, 0)
    zbase = sid * ZCOPY

    def zero_acc():
        for q in range(0, ZCOPY, ZROWS):
            n = min(ZROWS, ZCOPY - q)
            pltpu.sync_copy(zbuf.at[pl.ds(0, n)],
                            acc.at[pl.ds(zbase + q, n)])

    zero_acc()

    zi = jnp.zeros((L,), jnp.int32)
    iota = lax.broadcasted_iota(jnp.int32, (L,), 0)
    dump_lo = jnp.full((L,), DUMP, jnp.int32) + iota

    lo16 = jnp.full((L,), lo, jnp.int32)
    q16 = jnp.full((L,), Q, jnp.int32)
    one16 = jnp.full((L,), 1, jnp.int32)
    top16 = jnp.full((L,), CAP - 1, jnp.int32)

    @plsc.parallel_loop(
        0, NVEC, unroll=4,
        carry=(jnp.zeros((L,), jnp.int32), jnp.zeros((L,), jnp.int32)))
    def compact(i, ptrs):
        p0, p1 = ptrs
        s16 = src_v[pl.ds(i * L, L)]
        d16 = dst_v[pl.ds(i * L, L)]
        dl = d16 - lo16
        m0 = (dl >= 0) & (dl < q16)
        dl1 = dl - q16
        m1 = (dl1 >= 0) & (dl1 < q16)
        mi0 = jnp.where(m0, one16, zi)
        mi1 = jnp.where(m1, one16, zi)
        pos0 = p0 + plsc.cumsum(mi0) - mi0
        pos1 = top16 - (p1 + plsc.cumsum(mi1) - mi1)
        plsc.store_scatter(srcc, [pos0], s16, mask=m0)
        plsc.store_scatter(srcc, [pos1], s16, mask=m1)
        e0 = dl + dl
        e1 = dl1 + dl1
        plsc.store_scatter(dstc, [pos0 + pos0], e0, mask=m0)
        plsc.store_scatter(dstc, [pos0 + pos0 + one16], e0 + one16, mask=m0)
        plsc.store_scatter(dstc, [pos1 + pos1], e1, mask=m1)
        plsc.store_scatter(dstc, [pos1 + pos1 + one16], e1 + one16, mask=m1)
        return (p0 + plsc.all_reduce_population_count(m0),
                p1 + plsc.all_reduce_population_count(m1))

    ptr0, ptr1 = compact
    cnt_v[pl.ds(0, L)] = ptr0
    cnt_v[pl.ds(L, L)] = ptr1
    cnt0 = cnt_v[pl.ds(0, L)][0]
    cnt1 = cnt_v[pl.ds(L, L)][0]

    c0_16 = jnp.full((L,), cnt0, jnp.int32) + iota
    c1_16 = jnp.full((L,), CAP - cnt1 - NBUF * G, jnp.int32) + iota
    for k in range(NBUF * G // L):
        plsc.store_scatter(srcc, [c0_16 + k * L], zi)
        plsc.store_scatter(srcc, [c1_16 + k * L], zi)
    d0_16 = c0_16 + c0_16 - iota
    d1_16 = jnp.full((L,), 2 * (CAP - cnt1) - 2 * NBUF * G, jnp.int32) + iota
    for k in range(2 * NBUF * G // L):
        plsc.store_scatter(dstc, [d0_16 + k * L], dump_lo + (k % 2) * L)
        plsc.store_scatter(dstc, [d1_16 + k * L], dump_lo + (k % 2) * L)

    plsc.subcore_barrier()

    def off0(c):
        return c * G

    def off1(c):
        return CAP - G - c * G

    nchn0 = (cnt0 + (NBUF * G - 1)) // (NBUF * G)
    nchn1 = (cnt1 + (NBUF * G - 1)) // (NBUF * G)

    def start(off, c, b):
        pltpu.async_copy(
            x_hbm.at[srcc.at[pl.ds(off(c), G)]], gbuf.at[b], sems[b])

    def drain(b):
        pltpu.make_async_copy(
            x_hbm.at[pl.ds(0, G)], gbuf.at[b], sems[b]).wait()

    def scatter(off, c, b):
        idx = dstc.at[pl.ds(2 * off(c), 2 * G)]
        pltpu.sync_copy(gbuf.at[b].reshape(2 * G, DH),
                        acc.at[idx], add=True)

    def prologue(off, nchn):
        @pl.when(nchn > 0)
        def _():
            for b in range(NBUF):
                start(off, b, b)

    def hot_loop(off, nchn):
        def chunk_group(cc, carry):
            for b in range(NBUF):
                drain(b)
                scatter(off, NBUF * cc + b, b)

                @pl.when(cc + 1 < nchn)
                def _():
                    start(off, NBUF * (cc + 1) + b, b)

            return carry

        lax.fori_loop(0, nchn, chunk_group, 0)

    def copy_out(p):
        qlo2 = 2 * (lo + p * Q)

        def out_chunk(k, carry):
            j = sid + k * NS

            @pl.when(j < OUT_CHUNKS)
            def _():
                pltpu.async_copy(
                    acc.at[pl.ds(j * 8, 8)],
                    out_hbm.at[pl.ds(qlo2 + j * 8, 8)],
                    sem3)

            return carry

        def out_wait(k, carry):
            j = sid + k * NS

            @pl.when(j < OUT_CHUNKS)
            def _():
                pltpu.make_async_copy(
                    acc.at[pl.ds(0, 8)],
                    out_hbm.at[pl.ds(qlo2, 8)],
                    sem3).wait()

            return carry

        nk = (OUT_CHUNKS + NS - 1) // NS
        lax.fori_loop(0, nk, out_chunk, 0)
        lax.fori_loop(0, nk, out_wait, 0)

    prologue(off0, nchn0)
    hot_loop(off0, nchn0)
    prologue(off1, nchn1)

    plsc.subcore_barrier()
    copy_out(0)
    plsc.subcore_barrier()
    zero_acc()
    plsc.subcore_barrier()

    hot_loop(off1, nchn1)
    plsc.subcore_barrier()
    copy_out(1)


_seg_sum = pl.kernel(
    _body,
    out_type=jax.ShapeDtypeStruct((2 * N_NODES, DH), jnp.float32),
    mesh=plsc.VectorSubcoreMesh(
        core_axis_name="c", subcore_axis_name="s",
        num_cores=NC, num_subcores=NS),
    compiler_params=pltpu.CompilerParams(needs_layout_passes=False),
    scratch_types=[
        pltpu.VMEM((E_T,), jnp.int32),
        pltpu.VMEM((E_T,), jnp.int32),
        pltpu.VMEM((CAP,), jnp.int32),
        pltpu.VMEM((2 * CAP,), jnp.int32),
        pltpu.VMEM((NBUF, G, 2, DH), jnp.float32),
        pltpu.VMEM((ZROWS, DH), jnp.float32),
        pltpu.VMEM((2 * L,), jnp.int32),
        pltpu.VMEM_SHARED((ACC_ROWS, DH), jnp.float32),
        pltpu.SemaphoreType.DMA,
        pltpu.SemaphoreType.DMA,
        pltpu.SemaphoreType.DMA,
        pltpu.SemaphoreType.DMA,
        pltpu.SemaphoreType.DMA,
        pltpu.SemaphoreType.DMA,
        pltpu.SemaphoreType.DMA,
        pltpu.SemaphoreType.DMA,
        pltpu.SemaphoreType.DMA,
    ],
)


@jax.jit
def kernel(edge_index, mask, x):
    del mask
    src = edge_index[0]
    dst = edge_index[1]
    out2 = _seg_sum(src, dst, x.reshape(N_NODES, 2, DH))
    return out2.reshape(N_NODES, D)


# speedup vs baseline: 1.3569x; 1.0147x over previous
import jax
import jax.numpy as jnp
from jax import lax
from jax.experimental import pallas as pl
from jax.experimental.pallas import tpu as pltpu
from jax.experimental.pallas import tpu_sc as plsc

N_NODES = 10000
N_EDGES = 160000
D = 256

NC = 2
NS = 16
L = 16

DH = D // 2
Q = N_NODES // 4
ACC_ROWS = 2 * Q + 120
DUMP = 2 * Q
E_T = N_EDGES // NS
NBUF = 4
G = 16
CAP = 10240
NVEC = E_T // L
NFILL = CAP // L
ZCOPY = ACC_ROWS // NS
ZROWS = 32
OUT_CHUNKS = 2 * Q // 8


def _body(src_hbm, dst_hbm, x_hbm, out_hbm,
          src_v, dst_v, srcc, dstc, gbuf, zbuf, cnt_v, acc,
          sg0, sg1, sg2, sg3, ss0, ss1, ss2, ss3, sem3):
    sems = (sg0, sg1, sg2, sg3, ss0, ss1, ss2, ss3)
    cid = lax.axis_index("c")
    sid = lax.axis_index("s")
    lo = cid * 2 * Q

    eload0 = pltpu.async_copy(src_hbm.at[pl.ds(sid * E_T, E_T)], src_v,
                              sems[0])
    eload1 = pltpu.async_copy(dst_hbm.at[pl.ds(sid * E_T, E_T)], dst_v,
                              sems[1])

    zf = jnp.zeros((L,), jnp.float32)

    def zero_row(r, carry):
        for j in range(DH // L):
            zbuf[r, pl.ds(j * L, L)] = zf
        return carry

    lax.fori_loop(0, ZROWS, zero_row, 0)
    zbase = sid * ZCOPY

    def zero_acc_fire():
        for q in range(0, ZCOPY, ZROWS):
            n = min(ZROWS, ZCOPY - q)
            pltpu.async_copy(zbuf.at[pl.ds(0, n)],
                             acc.at[pl.ds(zbase + q, n)], sem3)

    def zero_acc_drain():
        for q in range(0, ZCOPY, ZROWS):
            n = min(ZROWS, ZCOPY - q)
            pltpu.make_async_copy(zbuf.at[pl.ds(0, n)],
                                  acc.at[pl.ds(zbase + q, n)], sem3).wait()

    def zero_acc():
        zero_acc_fire()
        zero_acc_drain()

    zero_acc_fire()
    eload0.wait()
    eload1.wait()

    zi = jnp.zeros((L,), jnp.int32)
    iota = lax.broadcasted_iota(jnp.int32, (L,), 0)
    dump_lo = jnp.full((L,), DUMP, jnp.int32) + iota

    lo16 = jnp.full((L,), lo, jnp.int32)
    q16 = jnp.full((L,), Q, jnp.int32)
    one16 = jnp.full((L,), 1, jnp.int32)
    top16 = jnp.full((L,), CAP - 1, jnp.int32)

    @plsc.parallel_loop(
        0, NVEC, unroll=4,
        carry=(jnp.zeros((L,), jnp.int32), jnp.zeros((L,), jnp.int32)))
    def compact(i, ptrs):
        p0, p1 = ptrs
        s16 = src_v[pl.ds(i * L, L)]
        d16 = dst_v[pl.ds(i * L, L)]
        dl = d16 - lo16
        m0 = (dl >= 0) & (dl < q16)
        dl1 = dl - q16
        m1 = (dl1 >= 0) & (dl1 < q16)
        mi0 = jnp.where(m0, one16, zi)
        mi1 = jnp.where(m1, one16, zi)
        pos0 = p0 + plsc.cumsum(mi0) - mi0
        pos1 = top16 - (p1 + plsc.cumsum(mi1) - mi1)
        plsc.store_scatter(srcc, [pos0], s16, mask=m0)
        plsc.store_scatter(srcc, [pos1], s16, mask=m1)
        e0 = dl + dl
        e1 = dl1 + dl1
        plsc.store_scatter(dstc, [pos0 + pos0], e0, mask=m0)
        plsc.store_scatter(dstc, [pos0 + pos0 + one16], e0 + one16, mask=m0)
        plsc.store_scatter(dstc, [pos1 + pos1], e1, mask=m1)
        plsc.store_scatter(dstc, [pos1 + pos1 + one16], e1 + one16, mask=m1)
        return (p0 + plsc.all_reduce_population_count(m0),
                p1 + plsc.all_reduce_population_count(m1))

    ptr0, ptr1 = compact
    cnt_v[pl.ds(0, L)] = ptr0
    cnt_v[pl.ds(L, L)] = ptr1
    cnt0 = cnt_v[pl.ds(0, L)][0]
    cnt1 = cnt_v[pl.ds(L, L)][0]

    c0_16 = jnp.full((L,), cnt0, jnp.int32) + iota
    c1_16 = jnp.full((L,), CAP - cnt1 - NBUF * G, jnp.int32) + iota
    for k in range(NBUF * G // L):
        plsc.store_scatter(srcc, [c0_16 + k * L], zi)
        plsc.store_scatter(srcc, [c1_16 + k * L], zi)
    d0_16 = c0_16 + c0_16 - iota
    d1_16 = jnp.full((L,), 2 * (CAP - cnt1) - 2 * NBUF * G, jnp.int32) + iota
    for k in range(2 * NBUF * G // L):
        plsc.store_scatter(dstc, [d0_16 + k * L], dump_lo + (k % 2) * L)
        plsc.store_scatter(dstc, [d1_16 + k * L], dump_lo + (k % 2) * L)

    zero_acc_drain()
    plsc.subcore_barrier()

    def off0(c):
        return c * G

    def off1(c):
        return CAP - G - c * G

    nchn0 = (cnt0 + (NBUF * G - 1)) // (NBUF * G)
    nchn1 = (cnt1 + (NBUF * G - 1)) // (NBUF * G)

    def start(off, c, b):
        pltpu.async_copy(
            x_hbm.at[srcc.at[pl.ds(off(c), G)]], gbuf.at[b], sems[b])

    def drain(b):
        pltpu.make_async_copy(
            x_hbm.at[pl.ds(0, G)], gbuf.at[b], sems[b]).wait()

    def scatter(off, c, b):
        idx = dstc.at[pl.ds(2 * off(c), 2 * G)]
        pltpu.sync_copy(gbuf.at[b].reshape(2 * G, DH),
                        acc.at[idx], add=True)

    def prologue(off, nchn):
        @pl.when(nchn > 0)
        def _():
            for b in range(NBUF):
                start(off, b, b)

    def hot_loop(off, nchn):
        def chunk_group(cc, carry):
            for b in range(NBUF):
                drain(b)
                scatter(off, NBUF * cc + b, b)

                @pl.when(cc + 1 < nchn)
                def _():
                    start(off, NBUF * (cc + 1) + b, b)

            return carry

        lax.fori_loop(0, nchn, chunk_group, 0)

    def copy_out(p):
        qlo2 = 2 * (lo + p * Q)

        def out_chunk(k, carry):
            j = sid + k * NS

            @pl.when(j < OUT_CHUNKS)
            def _():
                pltpu.async_copy(
                    acc.at[pl.ds(j * 8, 8)],
                    out_hbm.at[pl.ds(qlo2 + j * 8, 8)],
                    sem3)

            return carry

        def out_wait(k, carry):
            j = sid + k * NS

            @pl.when(j < OUT_CHUNKS)
            def _():
                pltpu.make_async_copy(
                    acc.at[pl.ds(0, 8)],
                    out_hbm.at[pl.ds(qlo2, 8)],
                    sem3).wait()

            return carry

        nk = (OUT_CHUNKS + NS - 1) // NS
        lax.fori_loop(0, nk, out_chunk, 0)
        lax.fori_loop(0, nk, out_wait, 0)

    prologue(off0, nchn0)
    hot_loop(off0, nchn0)
    prologue(off1, nchn1)

    plsc.subcore_barrier()
    copy_out(0)
    plsc.subcore_barrier()
    zero_acc()
    plsc.subcore_barrier()

    hot_loop(off1, nchn1)
    plsc.subcore_barrier()
    copy_out(1)


_seg_sum = pl.kernel(
    _body,
    out_type=jax.ShapeDtypeStruct((2 * N_NODES, DH), jnp.float32),
    mesh=plsc.VectorSubcoreMesh(
        core_axis_name="c", subcore_axis_name="s",
        num_cores=NC, num_subcores=NS),
    compiler_params=pltpu.CompilerParams(needs_layout_passes=False),
    scratch_types=[
        pltpu.VMEM((E_T,), jnp.int32),
        pltpu.VMEM((E_T,), jnp.int32),
        pltpu.VMEM((CAP,), jnp.int32),
        pltpu.VMEM((2 * CAP,), jnp.int32),
        pltpu.VMEM((NBUF, G, 2, DH), jnp.float32),
        pltpu.VMEM((ZROWS, DH), jnp.float32),
        pltpu.VMEM((2 * L,), jnp.int32),
        pltpu.VMEM_SHARED((ACC_ROWS, DH), jnp.float32),
        pltpu.SemaphoreType.DMA,
        pltpu.SemaphoreType.DMA,
        pltpu.SemaphoreType.DMA,
        pltpu.SemaphoreType.DMA,
        pltpu.SemaphoreType.DMA,
        pltpu.SemaphoreType.DMA,
        pltpu.SemaphoreType.DMA,
        pltpu.SemaphoreType.DMA,
        pltpu.SemaphoreType.DMA,
    ],
)


@jax.jit
def kernel(edge_index, mask, x):
    del mask
    src = edge_index[0]
    dst = edge_index[1]
    out2 = _seg_sum(src, dst, x.reshape(N_NODES, 2, DH))
    return out2.reshape(N_NODES, D)


# speedup vs baseline: 1.3578x; 1.0006x over previous
import jax
import jax.numpy as jnp
from jax import lax
from jax.experimental import pallas as pl
from jax.experimental.pallas import tpu as pltpu
from jax.experimental.pallas import tpu_sc as plsc

N_NODES = 10000
N_EDGES = 160000
D = 256

NC = 2
NS = 16
L = 16

DH = D // 2
Q = N_NODES // 4
ACC_ROWS = 2 * Q + 120
DUMP = 2 * Q
E_T = N_EDGES // NS
NBUF = 4
G = 16
CAP = 10240
NVEC = E_T // L
ZCOPY = ACC_ROWS // NS
ZROWS = 32
OUT_CHUNKS = 2 * Q // 8


def _body(src_hbm, dst_hbm, x_hbm, out_hbm,
          src_v, dst_v, srcc, dstc, gbuf, zbuf, cnt_v, acc,
          sg0, sg1, sg2, sg3, ss0, ss1, ss2, ss3, sem3):
    sems = (sg0, sg1, sg2, sg3, ss0, ss1, ss2, ss3)
    cid = lax.axis_index("c")
    sid = lax.axis_index("s")
    lo = cid * 2 * Q

    eload0 = pltpu.async_copy(src_hbm.at[pl.ds(sid * E_T, E_T)], src_v,
                              sems[0])
    eload1 = pltpu.async_copy(dst_hbm.at[pl.ds(sid * E_T, E_T)], dst_v,
                              sems[1])

    zf = jnp.zeros((L,), jnp.float32)

    def zero_row(r, carry):
        for j in range(DH // L):
            zbuf[r, pl.ds(j * L, L)] = zf
        return carry

    lax.fori_loop(0, ZROWS, zero_row, 0)
    zbase = sid * ZCOPY

    def zero_acc_fire():
        for q in range(0, ZCOPY, ZROWS):
            n = min(ZROWS, ZCOPY - q)
            pltpu.async_copy(zbuf.at[pl.ds(0, n)],
                             acc.at[pl.ds(zbase + q, n)], sem3)

    def zero_acc_drain():
        for q in range(0, ZCOPY, ZROWS):
            n = min(ZROWS, ZCOPY - q)
            pltpu.make_async_copy(zbuf.at[pl.ds(0, n)],
                                  acc.at[pl.ds(zbase + q, n)], sem3).wait()

    def zero_acc():
        zero_acc_fire()
        zero_acc_drain()

    zero_acc_fire()
    eload0.wait()
    eload1.wait()

    zi = jnp.zeros((L,), jnp.int32)
    iota = lax.broadcasted_iota(jnp.int32, (L,), 0)
    dump_lo = jnp.full((L,), DUMP, jnp.int32) + iota

    lo16 = jnp.full((L,), lo, jnp.int32)
    q16 = jnp.full((L,), Q, jnp.int32)
    one16 = jnp.full((L,), 1, jnp.int32)
    top16 = jnp.full((L,), CAP - 1, jnp.int32)

    @plsc.parallel_loop(
        0, NVEC, unroll=4,
        carry=(jnp.zeros((L,), jnp.int32), jnp.zeros((L,), jnp.int32)))
    def compact(i, ptrs):
        p0, p1 = ptrs
        s16 = src_v[pl.ds(i * L, L)]
        d16 = dst_v[pl.ds(i * L, L)]
        dl = d16 - lo16
        m0 = (dl >= 0) & (dl < q16)
        dl1 = dl - q16
        m1 = (dl1 >= 0) & (dl1 < q16)
        mi0 = jnp.where(m0, one16, zi)
        mi1 = jnp.where(m1, one16, zi)
        pos0 = p0 + plsc.cumsum(mi0) - mi0
        pos1 = top16 - (p1 + plsc.cumsum(mi1) - mi1)
        plsc.store_scatter(srcc, [pos0], s16, mask=m0)
        plsc.store_scatter(srcc, [pos1], s16, mask=m1)
        e0 = dl + dl
        e1 = dl1 + dl1
        plsc.store_scatter(dstc, [pos0 + pos0], e0, mask=m0)
        plsc.store_scatter(dstc, [pos0 + pos0 + one16], e0 + one16, mask=m0)
        plsc.store_scatter(dstc, [pos1 + pos1], e1, mask=m1)
        plsc.store_scatter(dstc, [pos1 + pos1 + one16], e1 + one16, mask=m1)
        return (p0 + plsc.all_reduce_population_count(m0),
                p1 + plsc.all_reduce_population_count(m1))

    ptr0, ptr1 = compact
    cnt_v[pl.ds(0, L)] = ptr0
    cnt_v[pl.ds(L, L)] = ptr1
    cnt0 = cnt_v[pl.ds(0, L)][0]
    cnt1 = cnt_v[pl.ds(L, L)][0]

    c0_16 = jnp.full((L,), cnt0, jnp.int32) + iota
    c1_16 = jnp.full((L,), CAP - cnt1 - NBUF * G, jnp.int32) + iota
    for k in range(NBUF * G // L):
        plsc.store_scatter(srcc, [c0_16 + k * L], zi)
        plsc.store_scatter(srcc, [c1_16 + k * L], zi)
    d0_16 = c0_16 + c0_16 - iota
    d1_16 = jnp.full((L,), 2 * (CAP - cnt1) - 2 * NBUF * G, jnp.int32) + iota
    for k in range(2 * NBUF * G // L):
        plsc.store_scatter(dstc, [d0_16 + k * L], dump_lo + (k % 2) * L)
        plsc.store_scatter(dstc, [d1_16 + k * L], dump_lo + (k % 2) * L)

    zero_acc_drain()
    plsc.subcore_barrier()

    def off0(c):
        return c * G

    def off1(c):
        return CAP - G - c * G

    nchn0 = (cnt0 + (NBUF * G - 1)) // (NBUF * G)
    nchn1 = (cnt1 + (NBUF * G - 1)) // (NBUF * G)

    def start(off, c, b):
        pltpu.async_copy(
            x_hbm.at[srcc.at[pl.ds(off(c), G)]], gbuf.at[b], sems[b])

    def drain(b):
        pltpu.make_async_copy(
            x_hbm.at[pl.ds(0, G)], gbuf.at[b], sems[b]).wait()

    def scatter(off, c, b):
        idx = dstc.at[pl.ds(2 * off(c), 2 * G)]
        pltpu.sync_copy(gbuf.at[b].reshape(2 * G, DH),
                        acc.at[idx], add=True)

    def prologue(off, nchn):
        @pl.when(nchn > 0)
        def _():
            for b in range(NBUF):
                start(off, b, b)

    def hot_loop(off, nchn):
        def chunk_group(cc, carry):
            for b in range(NBUF):
                drain(b)
                scatter(off, NBUF * cc + b, b)

                @pl.when(cc + 1 < nchn)
                def _():
                    start(off, NBUF * (cc + 1) + b, b)

            return carry

        lax.fori_loop(0, nchn, chunk_group, 0)

    def copy_out(p):
        qlo2 = 2 * (lo + p * Q)

        def out_chunk(k, carry):
            j = sid + k * NS

            @pl.when(j < OUT_CHUNKS)
            def _():
                pltpu.async_copy(
                    acc.at[pl.ds(j * 8, 8)],
                    out_hbm.at[pl.ds(qlo2 + j * 8, 8)],
                    sem3)

            return carry

        def out_wait(k, carry):
            j = sid + k * NS

            @pl.when(j < OUT_CHUNKS)
            def _():
                pltpu.make_async_copy(
                    acc.at[pl.ds(0, 8)],
                    out_hbm.at[pl.ds(qlo2, 8)],
                    sem3).wait()

            return carry

        nk = (OUT_CHUNKS + NS - 1) // NS
        lax.fori_loop(0, nk, out_chunk, 0)
        lax.fori_loop(0, nk, out_wait, 0)

    prologue(off0, nchn0)
    hot_loop(off0, nchn0)
    prologue(off1, nchn1)

    plsc.subcore_barrier()
    copy_out(0)
    plsc.subcore_barrier()
    zero_acc()
    plsc.subcore_barrier()

    hot_loop(off1, nchn1)
    plsc.subcore_barrier()
    copy_out(1)


_seg_sum = pl.kernel(
    _body,
    out_type=jax.ShapeDtypeStruct((2 * N_NODES, DH), jnp.float32),
    mesh=plsc.VectorSubcoreMesh(
        core_axis_name="c", subcore_axis_name="s",
        num_cores=NC, num_subcores=NS),
    compiler_params=pltpu.CompilerParams(needs_layout_passes=False),
    scratch_types=[
        pltpu.VMEM((E_T,), jnp.int32),
        pltpu.VMEM((E_T,), jnp.int32),
        pltpu.VMEM((CAP,), jnp.int32),
        pltpu.VMEM((2 * CAP,), jnp.int32),
        pltpu.VMEM((NBUF, G, 2, DH), jnp.float32),
        pltpu.VMEM((ZROWS, DH), jnp.float32),
        pltpu.VMEM((2 * L,), jnp.int32),
        pltpu.VMEM_SHARED((ACC_ROWS, DH), jnp.float32),
        pltpu.SemaphoreType.DMA,
        pltpu.SemaphoreType.DMA,
        pltpu.SemaphoreType.DMA,
        pltpu.SemaphoreType.DMA,
        pltpu.SemaphoreType.DMA,
        pltpu.SemaphoreType.DMA,
        pltpu.SemaphoreType.DMA,
        pltpu.SemaphoreType.DMA,
        pltpu.SemaphoreType.DMA,
    ],
)


@jax.jit
def kernel(edge_index, mask, x):
    del mask
    src = edge_index[0]
    dst = edge_index[1]
    out2 = _seg_sum(src, dst, x.reshape(N_NODES, 2, DH))
    return out2.reshape(N_NODES, D)
